# baseline (speedup 1.0000x reference)
import functools

import jax
import jax.numpy as jnp
from jax import lax
from jax.experimental import pallas as pl
from jax.experimental.pallas import tpu as pltpu
from jax.experimental.pallas import tpu_sc as plsc

T = 2048
H = 1024
FF = 4096
E = 8
K = 2
M = 256
ROWS_CAP = ((K * T + E * (M - 1) + M - 1) // M) * M
NUM_BLOCKS = ROWS_CAP // M
FF_BLK = 2048
FFC = FF // FF_BLK


def _router_body(x_ref, wr_ref, logits_ref, dest0_ref, dest1_ref,
                 w0_ref, w1_ref, counts_ref):
    x = x_ref[...]
    wr = wr_ref[...]
    logits = jnp.dot(x, wr, preferred_element_type=jnp.float32)
    logits_ref[...] = logits

    mx = jnp.max(logits, axis=-1, keepdims=True)
    p = jnp.exp(logits - mx)
    p = p / jnp.sum(p, axis=-1, keepdims=True)

    iota_e = jax.lax.broadcasted_iota(jnp.int32, (T, E), 1)
    w1 = jnp.max(p, axis=-1, keepdims=True)
    e1 = jnp.min(jnp.where(p == w1, iota_e, E), axis=-1, keepdims=True)
    p2 = jnp.where(iota_e == e1, -jnp.inf, p)
    w2 = jnp.max(p2, axis=-1, keepdims=True)
    e2 = jnp.min(jnp.where(p2 == w2, iota_e, E), axis=-1, keepdims=True)
    s = w1 + w2
    w1n = w1 / s
    w2n = w2 / s

    one1 = (iota_e == e1).astype(jnp.float32)
    one2 = (iota_e == e2).astype(jnp.float32)
    choose = one1 + one2

    r = jax.lax.broadcasted_iota(jnp.int32, (T, T), 0)
    c = jax.lax.broadcasted_iota(jnp.int32, (T, T), 1)
    lt = (c <= r).astype(jnp.float32)
    cum = jnp.dot(lt, choose, preferred_element_type=jnp.float32)
    counts = cum[T - 1:T, :]
    cum_excl = cum - choose

    nblk = jnp.floor((counts + (M - 1)) / M)
    padded = nblk * M
    re = jax.lax.broadcasted_iota(jnp.int32, (E, E), 0)
    ce = jax.lax.broadcasted_iota(jnp.int32, (E, E), 1)
    strict = (re < ce).astype(jnp.float32)
    off = jnp.dot(padded, strict, preferred_element_type=jnp.float32)

    pos = off + cum_excl
    dest0 = jnp.sum(one1 * pos, axis=-1, keepdims=True)
    dest1 = jnp.sum(one2 * pos, axis=-1, keepdims=True)
    dest0_ref[...] = dest0.astype(jnp.int32)
    dest1_ref[...] = dest1.astype(jnp.int32)
    w0_ref[...] = jnp.broadcast_to(w1n, (T, 128))
    w1_ref[...] = jnp.broadcast_to(w2n, (T, 128))
    counts_ref[...] = counts.astype(jnp.int32)


def _router(x, wr):
    return pl.pallas_call(
        _router_body,
        out_shape=(
            jax.ShapeDtypeStruct((T, E), jnp.float32),
            jax.ShapeDtypeStruct((T, 1), jnp.int32),
            jax.ShapeDtypeStruct((T, 1), jnp.int32),
            jax.ShapeDtypeStruct((T, 128), jnp.float32),
            jax.ShapeDtypeStruct((T, 128), jnp.float32),
            jax.ShapeDtypeStruct((1, E), jnp.int32),
        ),
    )(x, wr)


def _stage1_body(be_ref, na_ref, x_ref, wg_ref, bg_ref, wu_ref, bu_ref, h_ref):
    b = pl.program_id(1)

    @pl.when(b < na_ref[0])
    def _():
        x = x_ref[...].astype(jnp.bfloat16)
        g = jnp.dot(x, wg_ref[0].astype(jnp.bfloat16),
                    preferred_element_type=jnp.float32) + bg_ref[0]
        u = jnp.dot(x, wu_ref[0].astype(jnp.bfloat16),
                    preferred_element_type=jnp.float32) + bu_ref[0]
        h_ref[...] = g * jax.nn.sigmoid(g) * u


def _stage1(block_expert, num_active, xs, wg, bg, wu, bu):
    grid = (FFC, NUM_BLOCKS)
    return pl.pallas_call(
        _stage1_body,
        grid_spec=pltpu.PrefetchScalarGridSpec(
            num_scalar_prefetch=2,
            grid=grid,
            in_specs=[
                pl.BlockSpec((M, H), lambda f, b, be, na: (b, 0)),
                pl.BlockSpec((1, H, FF_BLK), lambda f, b, be, na: (be[b], 0, f)),
                pl.BlockSpec((1, 1, FF_BLK), lambda f, b, be, na: (be[b], 0, f)),
                pl.BlockSpec((1, H, FF_BLK), lambda f, b, be, na: (be[b], 0, f)),
                pl.BlockSpec((1, 1, FF_BLK), lambda f, b, be, na: (be[b], 0, f)),
            ],
            out_specs=pl.BlockSpec((M, FF_BLK), lambda f, b, be, na: (b, f)),
        ),
        out_shape=jax.ShapeDtypeStruct((ROWS_CAP, FF), jnp.float32),
        compiler_params=pltpu.CompilerParams(
            dimension_semantics=("arbitrary", "arbitrary"),
            vmem_limit_bytes=100 * 1024 * 1024,
        ),
    )(block_expert, num_active, xs, wg, bg, wu, bu)


def _stage2_body(be_ref, na_ref, h_ref, wd_ref, bd_ref, ws_ref, y_ref):
    b = pl.program_id(0)

    @pl.when(b < na_ref[0])
    def _():
        h = h_ref[...].astype(jnp.bfloat16)
        y = jnp.dot(h, wd_ref[0].astype(jnp.bfloat16),
                    preferred_element_type=jnp.float32) + bd_ref[0]
        y_ref[...] = y * ws_ref[:, 0:1]


def _stage2(block_expert, num_active, hs, wd, bd, ws):
    return pl.pallas_call(
        _stage2_body,
        grid_spec=pltpu.PrefetchScalarGridSpec(
            num_scalar_prefetch=2,
            grid=(NUM_BLOCKS,),
            in_specs=[
                pl.BlockSpec((M, FF), lambda b, be, na: (b, 0)),
                pl.BlockSpec((1, FF, H), lambda b, be, na: (be[b], 0, 0)),
                pl.BlockSpec((1, 1, H), lambda b, be, na: (be[b], 0, 0)),
                pl.BlockSpec((M, 128), lambda b, be, na: (b, 0)),
            ],
            out_specs=pl.BlockSpec((M, H), lambda b, be, na: (b, 0)),
        ),
        out_shape=jax.ShapeDtypeStruct((ROWS_CAP, H), jnp.float32),
        compiler_params=pltpu.CompilerParams(
            dimension_semantics=("arbitrary",),
            vmem_limit_bytes=100 * 1024 * 1024,
        ),
    )(block_expert, num_active, hs, wd, bd, ws)


_SC_MESH = plsc.VectorSubcoreMesh(
    core_axis_name="c", subcore_axis_name="s", num_cores=2, num_subcores=16)
_NW = 32
_TPW = T // _NW


def _dispatch_sc(x, dest0, dest1, w0b, w1b):
    @functools.partial(
        pl.kernel,
        mesh=_SC_MESH,
        out_type=(
            jax.ShapeDtypeStruct((ROWS_CAP, H), jnp.float32),
            jax.ShapeDtypeStruct((ROWS_CAP, 128), jnp.float32),
        ),
        scratch_types=[
            pltpu.VMEM((_TPW,), jnp.int32),
            pltpu.VMEM((_TPW,), jnp.int32),
            pltpu.VMEM((_TPW, H), jnp.float32),
            pltpu.VMEM((_TPW, 128), jnp.float32),
            pltpu.VMEM((_TPW, 128), jnp.float32),
            pltpu.SemaphoreType.DMA,
        ],
    )
    def run(x_hbm, d0_hbm, d1_hbm, w0_hbm, w1_hbm, xs_hbm, ws_hbm,
            idx0_v, idx1_v, xbuf, wbuf0, wbuf1, sem):
        wid = lax.axis_index("s") * 2 + lax.axis_index("c")
        base = wid * _TPW
        pltpu.sync_copy(d0_hbm.at[pl.ds(base, _TPW)], idx0_v)
        pltpu.sync_copy(d1_hbm.at[pl.ds(base, _TPW)], idx1_v)
        pltpu.sync_copy(x_hbm.at[pl.ds(base, _TPW)], xbuf)
        pltpu.sync_copy(w0_hbm.at[pl.ds(base, _TPW)], wbuf0)
        pltpu.sync_copy(w1_hbm.at[pl.ds(base, _TPW)], wbuf1)
        c1 = pltpu.async_copy(xbuf, xs_hbm.at[idx0_v], sem)
        c2 = pltpu.async_copy(xbuf, xs_hbm.at[idx1_v], sem)
        c3 = pltpu.async_copy(wbuf0, ws_hbm.at[idx0_v], sem)
        c4 = pltpu.async_copy(wbuf1, ws_hbm.at[idx1_v], sem)
        c1.wait()
        c2.wait()
        c3.wait()
        c4.wait()

    return run(x, dest0, dest1, w0b, w1b)


_CC = 32


def _combine_sc(ys, dest0, dest1):
    @functools.partial(
        pl.kernel,
        mesh=_SC_MESH,
        out_type=jax.ShapeDtypeStruct((T, H), jnp.float32),
        scratch_types=[
            pltpu.VMEM((_CC,), jnp.int32),
            pltpu.VMEM((_CC,), jnp.int32),
            pltpu.VMEM((_CC, H), jnp.float32),
            pltpu.VMEM((_CC, H), jnp.float32),
            pltpu.SemaphoreType.DMA,
        ],
    )
    def run(ys_hbm, d0_hbm, d1_hbm, out_hbm, idx0_v, idx1_v, buf0, buf1, sem):
        wid = lax.axis_index("s") * 2 + lax.axis_index("c")
        for cc in range(_TPW // _CC):
            base = wid * _TPW + cc * _CC
            pltpu.sync_copy(d0_hbm.at[pl.ds(base, _CC)], idx0_v)
            pltpu.sync_copy(d1_hbm.at[pl.ds(base, _CC)], idx1_v)
            g0 = pltpu.async_copy(ys_hbm.at[idx0_v], buf0, sem)
            g1 = pltpu.async_copy(ys_hbm.at[idx1_v], buf1, sem)
            g0.wait()
            g1.wait()

            def addrow(r, _):
                def addcol(c, __):
                    sl = pl.ds(c * 16, 16)
                    buf0[r, sl] = buf0[r, sl] + buf1[r, sl]
                    return __

                return lax.fori_loop(0, H // 16, addcol, _, unroll=8)

            lax.fori_loop(0, _CC, addrow, 0)
            pltpu.sync_copy(buf0, out_hbm.at[pl.ds(base, _CC)])

    return run(ys, dest0, dest1)


def kernel(X, Wr, Wg, bg, Wu, bu, Wd, bd):
    bsz, seq, hidden = X.shape
    x = X.reshape(T, H)

    logits, dest0, dest1, w0b, w1b, counts = _router(x, Wr)
    dest0 = dest0[:, 0]
    dest1 = dest1[:, 0]

    cnt = counts[0]
    nblk = (cnt + (M - 1)) // M
    cum = jnp.cumsum(nblk)
    used = cum[E - 1]
    b_ids = jnp.arange(NUM_BLOCKS, dtype=jnp.int32)
    bq = jnp.minimum(b_ids, used - 1)
    block_expert = jnp.searchsorted(cum, bq, side="right").astype(jnp.int32)
    num_active = used.reshape(1).astype(jnp.int32)

    xs, ws = _dispatch_sc(x, dest0, dest1, w0b, w1b)

    hs = _stage1(block_expert, num_active, xs, Wg,
                 bg.reshape(E, 1, FF), Wu, bu.reshape(E, 1, FF))
    ys = _stage2(block_expert, num_active, hs, Wd, bd.reshape(E, 1, H), ws)

    out = _combine_sc(ys, dest0, dest1)
    return out.reshape(bsz, seq, hidden), logits


# speedup vs baseline: 1.0527x; 1.0527x over previous
import functools

import jax
import jax.numpy as jnp
from jax import lax
from jax.experimental import pallas as pl
from jax.experimental.pallas import tpu as pltpu
from jax.experimental.pallas import tpu_sc as plsc

T = 2048
H = 1024
FF = 4096
E = 8
K = 2
M = 256
ROWS_CAP = ((K * T + E * (M - 1) + M - 1) // M) * M
NUM_BLOCKS = ROWS_CAP // M
FF_BLK = 2048
FFC = FF // FF_BLK


def _router_body(x_ref, wr_ref, logits_ref, dest0_ref, dest1_ref,
                 w0_ref, w1_ref, counts_ref):
    x = x_ref[...]
    wr = wr_ref[...]
    logits = jnp.dot(x, wr, preferred_element_type=jnp.float32)
    logits_ref[...] = logits

    mx = jnp.max(logits, axis=-1, keepdims=True)
    p = jnp.exp(logits - mx)
    p = p / jnp.sum(p, axis=-1, keepdims=True)

    iota_e = jax.lax.broadcasted_iota(jnp.int32, (T, E), 1)
    w1 = jnp.max(p, axis=-1, keepdims=True)
    e1 = jnp.min(jnp.where(p == w1, iota_e, E), axis=-1, keepdims=True)
    p2 = jnp.where(iota_e == e1, -jnp.inf, p)
    w2 = jnp.max(p2, axis=-1, keepdims=True)
    e2 = jnp.min(jnp.where(p2 == w2, iota_e, E), axis=-1, keepdims=True)
    s = w1 + w2
    w1n = w1 / s
    w2n = w2 / s

    one1 = (iota_e == e1).astype(jnp.float32)
    one2 = (iota_e == e2).astype(jnp.float32)
    choose = one1 + one2

    r = jax.lax.broadcasted_iota(jnp.int32, (T, T), 0)
    c = jax.lax.broadcasted_iota(jnp.int32, (T, T), 1)
    lt = (c <= r).astype(jnp.float32)
    cum = jnp.dot(lt, choose, preferred_element_type=jnp.float32)
    counts = cum[T - 1:T, :]
    cum_excl = cum - choose

    nblk = jnp.floor((counts + (M - 1)) / M)
    padded = nblk * M
    re = jax.lax.broadcasted_iota(jnp.int32, (E, E), 0)
    ce = jax.lax.broadcasted_iota(jnp.int32, (E, E), 1)
    strict = (re < ce).astype(jnp.float32)
    off = jnp.dot(padded, strict, preferred_element_type=jnp.float32)

    pos = off + cum_excl
    dest0 = jnp.sum(one1 * pos, axis=-1, keepdims=True)
    dest1 = jnp.sum(one2 * pos, axis=-1, keepdims=True)
    dest0_ref[...] = dest0.astype(jnp.int32)
    dest1_ref[...] = dest1.astype(jnp.int32)
    w0_ref[...] = jnp.broadcast_to(w1n, (T, 128))
    w1_ref[...] = jnp.broadcast_to(w2n, (T, 128))
    counts_ref[...] = counts.astype(jnp.int32)


def _router(x, wr):
    return pl.pallas_call(
        _router_body,
        out_shape=(
            jax.ShapeDtypeStruct((T, E), jnp.float32),
            jax.ShapeDtypeStruct((T, 1), jnp.int32),
            jax.ShapeDtypeStruct((T, 1), jnp.int32),
            jax.ShapeDtypeStruct((T, 128), jnp.float32),
            jax.ShapeDtypeStruct((T, 128), jnp.float32),
            jax.ShapeDtypeStruct((1, E), jnp.int32),
        ),
    )(x, wr)


def _stage1_body(be_ref, na_ref, x_ref, wg_ref, bg_ref, wu_ref, bu_ref, h_ref):
    b = pl.program_id(1)

    @pl.when(b < na_ref[0])
    def _():
        x = x_ref[...].astype(jnp.bfloat16)
        g = jnp.dot(x, wg_ref[0].astype(jnp.bfloat16),
                    preferred_element_type=jnp.float32) + bg_ref[0]
        u = jnp.dot(x, wu_ref[0].astype(jnp.bfloat16),
                    preferred_element_type=jnp.float32) + bu_ref[0]
        h_ref[...] = (g * jax.nn.sigmoid(g) * u).astype(jnp.bfloat16)


def _stage1(block_expert, num_active, xs, wg, bg, wu, bu):
    grid = (FFC, NUM_BLOCKS)
    return pl.pallas_call(
        _stage1_body,
        grid_spec=pltpu.PrefetchScalarGridSpec(
            num_scalar_prefetch=2,
            grid=grid,
            in_specs=[
                pl.BlockSpec((M, H), lambda f, b, be, na: (b, 0)),
                pl.BlockSpec((1, H, FF_BLK), lambda f, b, be, na: (be[b], 0, f)),
                pl.BlockSpec((1, 1, FF_BLK), lambda f, b, be, na: (be[b], 0, f)),
                pl.BlockSpec((1, H, FF_BLK), lambda f, b, be, na: (be[b], 0, f)),
                pl.BlockSpec((1, 1, FF_BLK), lambda f, b, be, na: (be[b], 0, f)),
            ],
            out_specs=pl.BlockSpec((M, FF_BLK), lambda f, b, be, na: (b, f)),
        ),
        out_shape=jax.ShapeDtypeStruct((ROWS_CAP, FF), jnp.bfloat16),
        compiler_params=pltpu.CompilerParams(
            dimension_semantics=("arbitrary", "arbitrary"),
            vmem_limit_bytes=100 * 1024 * 1024,
        ),
    )(block_expert, num_active, xs, wg, bg, wu, bu)


def _stage2_body(be_ref, na_ref, h_ref, wd_ref, bd_ref, ws_ref, y_ref):
    b = pl.program_id(0)

    @pl.when(b < na_ref[0])
    def _():
        h = h_ref[...]
        y = jnp.dot(h, wd_ref[0].astype(jnp.bfloat16),
                    preferred_element_type=jnp.float32) + bd_ref[0]
        y_ref[...] = y * ws_ref[:, 0:1]


def _stage2(block_expert, num_active, hs, wd, bd, ws):
    return pl.pallas_call(
        _stage2_body,
        grid_spec=pltpu.PrefetchScalarGridSpec(
            num_scalar_prefetch=2,
            grid=(NUM_BLOCKS,),
            in_specs=[
                pl.BlockSpec((M, FF), lambda b, be, na: (b, 0)),
                pl.BlockSpec((1, FF, H), lambda b, be, na: (be[b], 0, 0)),
                pl.BlockSpec((1, 1, H), lambda b, be, na: (be[b], 0, 0)),
                pl.BlockSpec((M, 128), lambda b, be, na: (b, 0)),
            ],
            out_specs=pl.BlockSpec((M, H), lambda b, be, na: (b, 0)),
        ),
        out_shape=jax.ShapeDtypeStruct((ROWS_CAP, H), jnp.float32),
        compiler_params=pltpu.CompilerParams(
            dimension_semantics=("arbitrary",),
            vmem_limit_bytes=100 * 1024 * 1024,
        ),
    )(block_expert, num_active, hs, wd, bd, ws)


_SC_MESH = plsc.VectorSubcoreMesh(
    core_axis_name="c", subcore_axis_name="s", num_cores=2, num_subcores=16)
_NW = 32
_TPW = T // _NW


def _dispatch_sc(x, dest0, dest1, w0b, w1b):
    @functools.partial(
        pl.kernel,
        mesh=_SC_MESH,
        out_type=(
            jax.ShapeDtypeStruct((ROWS_CAP, H), jnp.float32),
            jax.ShapeDtypeStruct((ROWS_CAP, 128), jnp.float32),
        ),
        scratch_types=[
            pltpu.VMEM((_TPW,), jnp.int32),
            pltpu.VMEM((_TPW,), jnp.int32),
            pltpu.VMEM((_TPW, H), jnp.float32),
            pltpu.VMEM((_TPW, 128), jnp.float32),
            pltpu.VMEM((_TPW, 128), jnp.float32),
            pltpu.SemaphoreType.DMA,
        ],
    )
    def run(x_hbm, d0_hbm, d1_hbm, w0_hbm, w1_hbm, xs_hbm, ws_hbm,
            idx0_v, idx1_v, xbuf, wbuf0, wbuf1, sem):
        wid = lax.axis_index("s") * 2 + lax.axis_index("c")
        base = wid * _TPW
        pltpu.sync_copy(d0_hbm.at[pl.ds(base, _TPW)], idx0_v)
        pltpu.sync_copy(d1_hbm.at[pl.ds(base, _TPW)], idx1_v)
        pltpu.sync_copy(x_hbm.at[pl.ds(base, _TPW)], xbuf)
        pltpu.sync_copy(w0_hbm.at[pl.ds(base, _TPW)], wbuf0)
        pltpu.sync_copy(w1_hbm.at[pl.ds(base, _TPW)], wbuf1)
        c1 = pltpu.async_copy(xbuf, xs_hbm.at[idx0_v], sem)
        c2 = pltpu.async_copy(xbuf, xs_hbm.at[idx1_v], sem)
        c3 = pltpu.async_copy(wbuf0, ws_hbm.at[idx0_v], sem)
        c4 = pltpu.async_copy(wbuf1, ws_hbm.at[idx1_v], sem)
        c1.wait()
        c2.wait()
        c3.wait()
        c4.wait()

    return run(x, dest0, dest1, w0b, w1b)


_CC = 32


def _combine_sc(ys, dest0, dest1):
    @functools.partial(
        pl.kernel,
        mesh=_SC_MESH,
        out_type=jax.ShapeDtypeStruct((T, H), jnp.float32),
        scratch_types=[
            pltpu.VMEM((_CC,), jnp.int32),
            pltpu.VMEM((_CC,), jnp.int32),
            pltpu.VMEM((_CC, H), jnp.float32),
            pltpu.VMEM((_CC, H), jnp.float32),
            pltpu.SemaphoreType.DMA,
        ],
    )
    def run(ys_hbm, d0_hbm, d1_hbm, out_hbm, idx0_v, idx1_v, buf0, buf1, sem):
        wid = lax.axis_index("s") * 2 + lax.axis_index("c")
        for cc in range(_TPW // _CC):
            base = wid * _TPW + cc * _CC
            pltpu.sync_copy(d0_hbm.at[pl.ds(base, _CC)], idx0_v)
            pltpu.sync_copy(d1_hbm.at[pl.ds(base, _CC)], idx1_v)
            g0 = pltpu.async_copy(ys_hbm.at[idx0_v], buf0, sem)
            g1 = pltpu.async_copy(ys_hbm.at[idx1_v], buf1, sem)
            g0.wait()
            g1.wait()

            def addrow(r, _):
                def addcol(c, __):
                    sl = pl.ds(c * 16, 16)
                    buf0[r, sl] = buf0[r, sl] + buf1[r, sl]
                    return __

                return lax.fori_loop(0, H // 16, addcol, _, unroll=8)

            lax.fori_loop(0, _CC, addrow, 0)
            pltpu.sync_copy(buf0, out_hbm.at[pl.ds(base, _CC)])

    return run(ys, dest0, dest1)


def kernel(X, Wr, Wg, bg, Wu, bu, Wd, bd):
    bsz, seq, hidden = X.shape
    x = X.reshape(T, H)

    logits, dest0, dest1, w0b, w1b, counts = _router(x, Wr)
    dest0 = dest0[:, 0]
    dest1 = dest1[:, 0]

    cnt = counts[0]
    nblk = (cnt + (M - 1)) // M
    cum = jnp.cumsum(nblk)
    used = cum[E - 1]
    b_ids = jnp.arange(NUM_BLOCKS, dtype=jnp.int32)
    bq = jnp.minimum(b_ids, used - 1)
    block_expert = jnp.searchsorted(cum, bq, side="right").astype(jnp.int32)
    num_active = used.reshape(1).astype(jnp.int32)

    xs, ws = _dispatch_sc(x, dest0, dest1, w0b, w1b)

    hs = _stage1(block_expert, num_active, xs, Wg,
                 bg.reshape(E, 1, FF), Wu, bu.reshape(E, 1, FF))
    ys = _stage2(block_expert, num_active, hs, Wd, bd.reshape(E, 1, H), ws)

    out = _combine_sc(ys, dest0, dest1)
    return out.reshape(bsz, seq, hidden), logits
